# baseline (speedup 1.0000x reference)
import functools

import jax
import jax.numpy as jnp
from jax import lax
from jax.experimental import pallas as pl
from jax.experimental.pallas import tpu as pltpu
from jax.experimental.pallas import tpu_sc as plsc

LANES = 16
BLK = 128
HALF = 32
QPB = BLK // HALF
SUP = 4
NC, NS = 2, 16
NW = NC * NS


def _mm_body(h_ref, wl_ref, bl_ref, wh_ref, bth_ref, h3_ref, p_ref):
    hb = h_ref[...]
    h3_ref[...] = jnp.dot(hb, wl_ref[...],
                          preferred_element_type=jnp.float32) + bl_ref[...]
    p_ref[...] = (jnp.dot(hb, wh_ref[...],
                          preferred_element_type=jnp.float32)
                  + bth_ref[...]).astype(jnp.bfloat16)


def _combine_body(n, h3_ref, parts_ref, eps_ref, out_ref):
    scale = 1.0 + eps_ref[0]
    sl = pl.ds(0, n)
    out_ref[...] = h3_ref[...] + scale * (parts_ref[0, sl] + parts_ref[1, sl])


def _make_sc_kernel(n_nodes, d, n_blocks_per_worker, n_acc_rows, zrows):
    mesh = plsc.VectorSubcoreMesh(core_axis_name="c", subcore_axis_name="s")
    nsup = n_blocks_per_worker // SUP
    npair = SUP // 2
    nch = d // LANES

    @functools.partial(
        pl.kernel,
        out_type=jax.ShapeDtypeStruct((NC, n_acc_rows, d), jnp.float32),
        mesh=mesh,
        scratch_types=[
            pltpu.VMEM((SUP, BLK), jnp.int32),
            pltpu.VMEM((SUP, BLK), jnp.int32),
            pltpu.VMEM((SUP * QPB, HALF), jnp.int32),
            pltpu.VMEM((3, SUP * BLK), jnp.float32),
            pltpu.VMEM((2 * BLK, d // 2), jnp.int32),
            pltpu.VMEM((2 * BLK, d // 2), jnp.int32),
            pltpu.VMEM((2 * HALF, d), jnp.float32),
            pltpu.VMEM((3, d), jnp.float32),
            pltpu.VMEM_SHARED((n_acc_rows, d), jnp.float32),
            pltpu.SemaphoreType.DMA,
            pltpu.SemaphoreType.DMA((2,)),
            pltpu.SemaphoreType.DMA((2,)),
        ],
        compiler_params=pltpu.CompilerParams(use_tc_tiling_on_sc=False),
    )
    def sc_kernel(p_hbm, p0_hbm, p1_hbm, si_hbm, ds_hbm, wd_hbm, z_hbm,
                  out_hbm, idx0_v, idx1_v, sidx_v, dsum_v,
                  r0buf, r1buf, out_v, wd_v, acc, sem_i, gsem, ssem):
        c = lax.axis_index("c")
        s = lax.axis_index("s")
        wid = c * NS + s

        pltpu.sync_copy(z_hbm, acc.at[pl.ds(s * zrows, zrows)])
        pltpu.sync_copy(wd_hbm, wd_v)
        plsc.subcore_barrier()

        def gathers(j, ph):
            sl = pl.ds(ph * BLK, BLK)
            pltpu.async_copy(p_hbm.at[idx0_v.at[j]], r0buf.at[sl],
                             gsem.at[ph])
            pltpu.async_copy(p_hbm.at[idx1_v.at[j]], r1buf.at[sl],
                             gsem.at[ph])

        def wait_gathers(j, ph):
            sl = pl.ds(ph * BLK, BLK)
            pltpu.make_async_copy(p_hbm.at[idx0_v.at[j]], r0buf.at[sl],
                                  gsem.at[ph]).wait()
            pltpu.make_async_copy(p_hbm.at[idx1_v.at[j]], r1buf.at[sl],
                                  gsem.at[ph]).wait()

        def wait_scatter(oh, hj):
            pltpu.make_async_copy(out_v.at[pl.ds(oh * HALF, HALF)],
                                  acc.at[sidx_v.at[hj]],
                                  ssem.at[oh]).wait()

        def compute_half(t, h, oh, ph):
            def group(g, carry):
                wch = [[wd_v[k, pl.ds(ch * LANES, LANES)]
                        for ch in range(nch)] for k in range(3)]
                goff = t * BLK + h * HALF + g * LANES
                dsv = [dsum_v[k, pl.ds(goff, LANES)] for k in range(3)]
                for el in range(LANES):
                    e = ph * BLK + h * HALF + g * LANES + el
                    o = oh * HALF + g * LANES + el
                    ds0, ds1, ds2 = dsv[0][el], dsv[1][el], dsv[2][el]
                    for c2 in range(nch // 2):
                        x0 = r0buf[e, pl.ds(c2 * LANES, LANES)]
                        x1 = r1buf[e, pl.ds(c2 * LANES, LANES)]
                        bc = lax.bitcast_convert_type
                        a0 = bc(x0 << 16, jnp.float32)
                        b0 = bc(x0 & jnp.int32(-65536), jnp.float32)
                        a1 = bc(x1 << 16, jnp.float32)
                        b1 = bc(x1 & jnp.int32(-65536), jnp.float32)
                        va = a0 + a1
                        va = va + ds0 * wch[0][2 * c2]
                        va = va + ds1 * wch[1][2 * c2]
                        va = va + ds2 * wch[2][2 * c2]
                        vb = b0 + b1
                        vb = vb + ds0 * wch[0][2 * c2 + 1]
                        vb = vb + ds1 * wch[1][2 * c2 + 1]
                        vb = vb + ds2 * wch[2][2 * c2 + 1]
                        out_v[o, pl.ds(c2 * 2 * LANES, LANES)] = (
                            jnp.maximum(va, 0.0))
                        out_v[o, pl.ds(c2 * 2 * LANES + LANES, LANES)] = (
                            jnp.maximum(vb, 0.0))
                return carry

            lax.fori_loop(0, HALF // LANES, group, 0, unroll=False)

        def superblock(sb, carry):
            @pl.when(sb > 0)
            def _():
                wait_scatter(0, QPB * SUP - 2)
                wait_scatter(1, QPB * SUP - 1)

            row0 = (wid * n_blocks_per_worker) + sb * SUP
            sb_global = wid * nsup + sb
            cps = [
                pltpu.async_copy(p0_hbm.at[pl.ds(row0, SUP)], idx0_v, sem_i),
                pltpu.async_copy(p1_hbm.at[pl.ds(row0, SUP)], idx1_v, sem_i),
                pltpu.async_copy(si_hbm.at[pl.ds(row0 * QPB, SUP * QPB)],
                                 sidx_v, sem_i),
                pltpu.async_copy(ds_hbm.at[sb_global], dsum_v, sem_i),
            ]
            for cp in cps:
                cp.wait()
            gathers(0, 0)

            def blk(t, carry2):
                ph = t % 2

                @pl.when(t < SUP - 1)
                def _():
                    gathers(t + 1, 1 - ph)

                wait_gathers(t, ph)

                def half(h, carry3):
                    oh = h % 2
                    @pl.when(QPB * t + h > 1)
                    def _():
                        wait_scatter(oh, QPB * t + h - 2)

                    compute_half(t, h, oh, ph)
                    pltpu.async_copy(
                        out_v.at[pl.ds(oh * HALF, HALF)],
                        acc.at[sidx_v.at[QPB * t + h]],
                        ssem.at[oh], add=True)
                    return carry3

                lax.fori_loop(0, QPB, half, 0, unroll=False)
                return carry2

            lax.fori_loop(0, SUP, blk, 0, unroll=False)
            return carry

        lax.fori_loop(0, nsup, superblock, 0, unroll=False)
        wait_scatter(0, QPB * SUP - 2)
        wait_scatter(1, QPB * SUP - 1)

        plsc.subcore_barrier()
        pltpu.sync_copy(acc.at[pl.ds(s * zrows, zrows)],
                        out_hbm.at[c, pl.ds(s * zrows, zrows)])

    return sc_kernel


def kernel(h, pairs_0, pairs_1, degrees_0, degrees_1, scatter_idx,
           W_lin, b_lin, W_t, b_t, eps):
    n, d_in = h.shape
    d_out = W_lin.shape[1]
    e = pairs_0.shape[0]

    iperm = jnp.arange(d_out).reshape(d_out // 32, 2, 16).transpose(
        0, 2, 1).reshape(d_out)
    w_h = W_t[:d_in][:, iperm]
    w_d = W_t[d_in:]
    h3, p_tab = pl.pallas_call(
        _mm_body,
        out_shape=(jax.ShapeDtypeStruct((n, d_out), jnp.float32),
                   jax.ShapeDtypeStruct((n, d_out), jnp.bfloat16)),
    )(h, W_lin, b_lin.reshape(1, d_out),
      w_h, (0.5 * b_t)[iperm].reshape(1, d_out))
    p_i32 = lax.bitcast_convert_type(
        p_tab.reshape(n, d_out // 2, 2), jnp.int32)

    chunk = NW * SUP * BLK
    e_pad = -(-e // chunk) * chunk
    pad = e_pad - e
    zrows = -(-(n + 1) // (NS * 8)) * 8
    n_dump = NS * zrows
    p0 = jnp.pad(pairs_0, (0, pad)).reshape(e_pad // BLK, BLK)
    p1 = jnp.pad(pairs_1, (0, pad)).reshape(e_pad // BLK, BLK)
    si = jnp.pad(scatter_idx, (0, pad), constant_values=n).reshape(
        e_pad // HALF, HALF)
    nsb = e_pad // (SUP * BLK)
    dsum = jnp.pad(degrees_0 + degrees_1, ((0, pad), (0, 0))).T.reshape(
        3, nsb, SUP * BLK).transpose(1, 0, 2)
    zeros = jnp.zeros((zrows, d_out), jnp.float32)

    sc = _make_sc_kernel(n, d_out, e_pad // BLK // NW, n_dump, zrows)
    partials = sc(p_i32, p0, p1, si, dsum, w_d, zeros)

    out = pl.pallas_call(
        functools.partial(_combine_body, n),
        in_specs=[pl.BlockSpec(memory_space=pltpu.VMEM),
                  pl.BlockSpec(memory_space=pltpu.VMEM),
                  pl.BlockSpec(memory_space=pltpu.SMEM)],
        out_shape=jax.ShapeDtypeStruct((n, d_out), jnp.float32),
    )(h3, partials, eps)
    return out


# speedup vs baseline: 1.1747x; 1.1747x over previous
import functools

import jax
import jax.numpy as jnp
from jax import lax
from jax.experimental import pallas as pl
from jax.experimental.pallas import tpu as pltpu
from jax.experimental.pallas import tpu_sc as plsc

LANES = 16
BLK = 64
HALF = 32
SUP = 16
NC, NS = 2, 16
NW = NC * NS


def _mm_body(h_ref, wl_ref, bl_ref, wh_ref, bth_ref, h3_ref, p_ref):
    hb = h_ref[...]
    h3_ref[...] = jnp.dot(hb, wl_ref[...],
                          preferred_element_type=jnp.float32) + bl_ref[...]
    p_ref[...] = (jnp.dot(hb, wh_ref[...],
                          preferred_element_type=jnp.float32)
                  + bth_ref[...]).astype(jnp.bfloat16)


def _combine_body(n, h3_ref, parts_ref, eps_ref, out_ref):
    scale = 1.0 + eps_ref[0]
    sl = pl.ds(0, n)
    out_ref[...] = h3_ref[...] + scale * (parts_ref[0, sl] + parts_ref[1, sl])


def _make_sc_kernel(n_nodes, d, n_blocks_per_worker, n_acc_rows, zrows):
    mesh = plsc.VectorSubcoreMesh(core_axis_name="c", subcore_axis_name="s")
    nsup = n_blocks_per_worker // SUP
    npair = SUP // 2
    nch = d // LANES

    @functools.partial(
        pl.kernel,
        out_type=jax.ShapeDtypeStruct((NC, n_acc_rows, d), jnp.float32),
        mesh=mesh,
        scratch_types=[
            pltpu.VMEM((SUP, BLK), jnp.int32),
            pltpu.VMEM((SUP, BLK), jnp.int32),
            pltpu.VMEM((SUP * 2, HALF), jnp.int32),
            pltpu.VMEM((3, SUP * BLK), jnp.float32),
            pltpu.VMEM((2 * BLK, d // 2), jnp.int32),
            pltpu.VMEM((2 * BLK, d // 2), jnp.int32),
            pltpu.VMEM((2 * HALF, d), jnp.float32),
            pltpu.VMEM((3, d), jnp.float32),
            pltpu.VMEM_SHARED((n_acc_rows, d), jnp.float32),
            pltpu.SemaphoreType.DMA,
            pltpu.SemaphoreType.DMA((2,)),
            pltpu.SemaphoreType.DMA((2,)),
        ],
        compiler_params=pltpu.CompilerParams(use_tc_tiling_on_sc=False),
    )
    def sc_kernel(p_hbm, p0_hbm, p1_hbm, si_hbm, ds_hbm, wd_hbm, z_hbm,
                  out_hbm, idx0_v, idx1_v, sidx_v, dsum_v,
                  r0buf, r1buf, out_v, wd_v, acc, sem_i, gsem, ssem):
        c = lax.axis_index("c")
        s = lax.axis_index("s")
        wid = c * NS + s

        pltpu.sync_copy(z_hbm, acc.at[pl.ds(s * zrows, zrows)])
        pltpu.sync_copy(wd_hbm, wd_v)
        plsc.subcore_barrier()

        def gathers(j, ph):
            sl = pl.ds(ph * BLK, BLK)
            pltpu.async_copy(p_hbm.at[idx0_v.at[j]], r0buf.at[sl],
                             gsem.at[ph])
            pltpu.async_copy(p_hbm.at[idx1_v.at[j]], r1buf.at[sl],
                             gsem.at[ph])

        def wait_gathers(j, ph):
            sl = pl.ds(ph * BLK, BLK)
            pltpu.make_async_copy(p_hbm.at[idx0_v.at[j]], r0buf.at[sl],
                                  gsem.at[ph]).wait()
            pltpu.make_async_copy(p_hbm.at[idx1_v.at[j]], r1buf.at[sl],
                                  gsem.at[ph]).wait()

        def wait_scatter(h, hj):
            pltpu.make_async_copy(out_v.at[pl.ds(h * HALF, HALF)],
                                  acc.at[sidx_v.at[hj]],
                                  ssem.at[h]).wait()

        def compute_half(t, h, ph):
            def group(g, carry):
                wch = [[wd_v[k, pl.ds(ch * LANES, LANES)]
                        for ch in range(nch)] for k in range(3)]
                goff = t * BLK + h * HALF + g * LANES
                dsv = [dsum_v[k, pl.ds(goff, LANES)] for k in range(3)]
                for el in range(LANES):
                    e = ph * BLK + h * HALF + g * LANES + el
                    o = h * HALF + g * LANES + el
                    ds0, ds1, ds2 = dsv[0][el], dsv[1][el], dsv[2][el]
                    for c2 in range(nch // 2):
                        x0 = r0buf[e, pl.ds(c2 * LANES, LANES)]
                        x1 = r1buf[e, pl.ds(c2 * LANES, LANES)]
                        bc = lax.bitcast_convert_type
                        a0 = bc(x0 << 16, jnp.float32)
                        b0 = bc(x0 & jnp.int32(-65536), jnp.float32)
                        a1 = bc(x1 << 16, jnp.float32)
                        b1 = bc(x1 & jnp.int32(-65536), jnp.float32)
                        va = a0 + a1
                        va = va + ds0 * wch[0][2 * c2]
                        va = va + ds1 * wch[1][2 * c2]
                        va = va + ds2 * wch[2][2 * c2]
                        vb = b0 + b1
                        vb = vb + ds0 * wch[0][2 * c2 + 1]
                        vb = vb + ds1 * wch[1][2 * c2 + 1]
                        vb = vb + ds2 * wch[2][2 * c2 + 1]
                        out_v[o, pl.ds(c2 * 2 * LANES, LANES)] = (
                            jnp.maximum(va, 0.0))
                        out_v[o, pl.ds(c2 * 2 * LANES + LANES, LANES)] = (
                            jnp.maximum(vb, 0.0))
                return carry

            lax.fori_loop(0, HALF // LANES, group, 0, unroll=False)

        def superblock(sb, carry):
            @pl.when(sb > 0)
            def _():
                wait_scatter(0, 2 * SUP - 2)
                wait_scatter(1, 2 * SUP - 1)

            row0 = (wid * n_blocks_per_worker) + sb * SUP
            sb_global = wid * nsup + sb
            cps = [
                pltpu.async_copy(p0_hbm.at[pl.ds(row0, SUP)], idx0_v, sem_i),
                pltpu.async_copy(p1_hbm.at[pl.ds(row0, SUP)], idx1_v, sem_i),
                pltpu.async_copy(si_hbm.at[pl.ds(row0 * 2, SUP * 2)], sidx_v,
                                 sem_i),
                pltpu.async_copy(ds_hbm.at[sb_global], dsum_v, sem_i),
            ]
            for cp in cps:
                cp.wait()
            gathers(0, 0)

            def blk(t, carry2):
                ph = t % 2

                @pl.when(t < SUP - 1)
                def _():
                    gathers(t + 1, 1 - ph)

                wait_gathers(t, ph)

                def half(h, carry3):
                    @pl.when(t > 0)
                    def _():
                        wait_scatter(h, 2 * t + h - 2)

                    compute_half(t, h, ph)
                    pltpu.async_copy(
                        out_v.at[pl.ds(h * HALF, HALF)],
                        acc.at[sidx_v.at[2 * t + h]],
                        ssem.at[h], add=True)
                    return carry3

                lax.fori_loop(0, 2, half, 0, unroll=False)
                return carry2

            lax.fori_loop(0, SUP, blk, 0, unroll=False)
            return carry

        lax.fori_loop(0, nsup, superblock, 0, unroll=False)
        wait_scatter(0, 2 * SUP - 2)
        wait_scatter(1, 2 * SUP - 1)

        plsc.subcore_barrier()
        pltpu.sync_copy(acc.at[pl.ds(s * zrows, zrows)],
                        out_hbm.at[c, pl.ds(s * zrows, zrows)])

    return sc_kernel


def kernel(h, pairs_0, pairs_1, degrees_0, degrees_1, scatter_idx,
           W_lin, b_lin, W_t, b_t, eps):
    n, d_in = h.shape
    d_out = W_lin.shape[1]
    e = pairs_0.shape[0]

    iperm = jnp.arange(d_out).reshape(d_out // 32, 2, 16).transpose(
        0, 2, 1).reshape(d_out)
    w_h = W_t[:d_in][:, iperm]
    w_d = W_t[d_in:]
    h3, p_tab = pl.pallas_call(
        _mm_body,
        out_shape=(jax.ShapeDtypeStruct((n, d_out), jnp.float32),
                   jax.ShapeDtypeStruct((n, d_out), jnp.bfloat16)),
    )(h, W_lin, b_lin.reshape(1, d_out),
      w_h, (0.5 * b_t)[iperm].reshape(1, d_out))
    p_i32 = lax.bitcast_convert_type(
        p_tab.reshape(n, d_out // 2, 2), jnp.int32)

    chunk = NW * SUP * BLK
    e_pad = -(-e // chunk) * chunk
    pad = e_pad - e
    zrows = -(-(n + 1) // (NS * 8)) * 8
    n_dump = NS * zrows
    p0 = jnp.pad(pairs_0, (0, pad)).reshape(e_pad // BLK, BLK)
    p1 = jnp.pad(pairs_1, (0, pad)).reshape(e_pad // BLK, BLK)
    si = jnp.pad(scatter_idx, (0, pad), constant_values=n).reshape(
        e_pad // HALF, HALF)
    nsb = e_pad // (SUP * BLK)
    dsum = jnp.pad(degrees_0 + degrees_1, ((0, pad), (0, 0))).T.reshape(
        3, nsb, SUP * BLK).transpose(1, 0, 2)
    zeros = jnp.zeros((zrows, d_out), jnp.float32)

    sc = _make_sc_kernel(n, d_out, e_pad // BLK // NW, n_dump, zrows)
    partials = sc(p_i32, p0, p1, si, dsum, w_d, zeros)

    out = pl.pallas_call(
        functools.partial(_combine_body, n),
        in_specs=[pl.BlockSpec(memory_space=pltpu.VMEM),
                  pl.BlockSpec(memory_space=pltpu.VMEM),
                  pl.BlockSpec(memory_space=pltpu.SMEM)],
        out_shape=jax.ShapeDtypeStruct((n, d_out), jnp.float32),
    )(h3, partials, eps)
    return out


# speedup vs baseline: 1.2234x; 1.0415x over previous
import functools

import jax
import jax.numpy as jnp
from jax import lax
from jax.experimental import pallas as pl
from jax.experimental.pallas import tpu as pltpu
from jax.experimental.pallas import tpu_sc as plsc

LANES = 16
BLK = 64
HALF = 32
SUP = 40
NC, NS = 2, 16
NW = NC * NS


def _mm_body(h_ref, wl_ref, bl_ref, wh_ref, bth_ref, h3_ref, p_ref):
    hb = h_ref[...]
    h3_ref[...] = jnp.dot(hb, wl_ref[...],
                          preferred_element_type=jnp.float32) + bl_ref[...]
    p_ref[...] = (jnp.dot(hb, wh_ref[...],
                          preferred_element_type=jnp.float32)
                  + bth_ref[...]).astype(jnp.bfloat16)


def _combine_body(n, h3_ref, parts_ref, eps_ref, out_ref):
    scale = 1.0 + eps_ref[0]
    sl = pl.ds(0, n)
    out_ref[...] = h3_ref[...] + scale * (parts_ref[0, sl] + parts_ref[1, sl])


def _make_sc_kernel(n_nodes, d, n_blocks_per_worker, n_acc_rows, zrows):
    mesh = plsc.VectorSubcoreMesh(core_axis_name="c", subcore_axis_name="s")
    nsup = n_blocks_per_worker // SUP
    npair = SUP // 2
    nch = d // LANES

    @functools.partial(
        pl.kernel,
        out_type=jax.ShapeDtypeStruct((NC, n_acc_rows, d), jnp.float32),
        mesh=mesh,
        scratch_types=[
            pltpu.VMEM((SUP, BLK), jnp.int32),
            pltpu.VMEM((SUP, BLK), jnp.int32),
            pltpu.VMEM((SUP * 2, HALF), jnp.int32),
            pltpu.VMEM((3, SUP * BLK), jnp.float32),
            pltpu.VMEM((2 * BLK, d // 2), jnp.int32),
            pltpu.VMEM((2 * BLK, d // 2), jnp.int32),
            pltpu.VMEM((2 * HALF, d), jnp.float32),
            pltpu.VMEM((3, d), jnp.float32),
            pltpu.VMEM_SHARED((n_acc_rows, d), jnp.float32),
            pltpu.SemaphoreType.DMA,
            pltpu.SemaphoreType.DMA((2,)),
            pltpu.SemaphoreType.DMA((2,)),
        ],
        compiler_params=pltpu.CompilerParams(use_tc_tiling_on_sc=False),
    )
    def sc_kernel(p_hbm, p0_hbm, p1_hbm, si_hbm, ds_hbm, wd_hbm, z_hbm,
                  out_hbm, idx0_v, idx1_v, sidx_v, dsum_v,
                  r0buf, r1buf, out_v, wd_v, acc, sem_i, gsem, ssem):
        c = lax.axis_index("c")
        s = lax.axis_index("s")
        wid = c * NS + s

        pltpu.sync_copy(z_hbm, acc.at[pl.ds(s * zrows, zrows)])
        pltpu.sync_copy(wd_hbm, wd_v)
        plsc.subcore_barrier()

        def gathers(j, ph):
            sl = pl.ds(ph * BLK, BLK)
            pltpu.async_copy(p_hbm.at[idx0_v.at[j]], r0buf.at[sl],
                             gsem.at[ph])
            pltpu.async_copy(p_hbm.at[idx1_v.at[j]], r1buf.at[sl],
                             gsem.at[ph])

        def wait_gathers(j, ph):
            sl = pl.ds(ph * BLK, BLK)
            pltpu.make_async_copy(p_hbm.at[idx0_v.at[j]], r0buf.at[sl],
                                  gsem.at[ph]).wait()
            pltpu.make_async_copy(p_hbm.at[idx1_v.at[j]], r1buf.at[sl],
                                  gsem.at[ph]).wait()

        def wait_scatter(h, hj):
            pltpu.make_async_copy(out_v.at[pl.ds(h * HALF, HALF)],
                                  acc.at[sidx_v.at[hj]],
                                  ssem.at[h]).wait()

        def compute_half(t, h, ph):
            def group(g, carry):
                wch = [[wd_v[k, pl.ds(ch * LANES, LANES)]
                        for ch in range(nch)] for k in range(3)]
                goff = t * BLK + h * HALF + g * LANES
                dsv = [dsum_v[k, pl.ds(goff, LANES)] for k in range(3)]
                for el in range(LANES):
                    e = ph * BLK + h * HALF + g * LANES + el
                    o = h * HALF + g * LANES + el
                    ds0, ds1, ds2 = dsv[0][el], dsv[1][el], dsv[2][el]
                    for c2 in range(nch // 2):
                        x0 = r0buf[e, pl.ds(c2 * LANES, LANES)]
                        x1 = r1buf[e, pl.ds(c2 * LANES, LANES)]
                        bc = lax.bitcast_convert_type
                        a0 = bc(x0 << 16, jnp.float32)
                        b0 = bc(x0 & jnp.int32(-65536), jnp.float32)
                        a1 = bc(x1 << 16, jnp.float32)
                        b1 = bc(x1 & jnp.int32(-65536), jnp.float32)
                        va = a0 + a1
                        va = va + ds0 * wch[0][2 * c2]
                        va = va + ds1 * wch[1][2 * c2]
                        va = va + ds2 * wch[2][2 * c2]
                        vb = b0 + b1
                        vb = vb + ds0 * wch[0][2 * c2 + 1]
                        vb = vb + ds1 * wch[1][2 * c2 + 1]
                        vb = vb + ds2 * wch[2][2 * c2 + 1]
                        out_v[o, pl.ds(c2 * 2 * LANES, LANES)] = (
                            jnp.maximum(va, 0.0))
                        out_v[o, pl.ds(c2 * 2 * LANES + LANES, LANES)] = (
                            jnp.maximum(vb, 0.0))
                return carry

            lax.fori_loop(0, HALF // LANES, group, 0, unroll=False)

        def superblock(sb, carry):
            @pl.when(sb > 0)
            def _():
                wait_scatter(0, 2 * SUP - 2)
                wait_scatter(1, 2 * SUP - 1)

            row0 = (wid * n_blocks_per_worker) + sb * SUP
            sb_global = wid * nsup + sb
            cps = [
                pltpu.async_copy(p0_hbm.at[pl.ds(row0, SUP)], idx0_v, sem_i),
                pltpu.async_copy(p1_hbm.at[pl.ds(row0, SUP)], idx1_v, sem_i),
                pltpu.async_copy(si_hbm.at[pl.ds(row0 * 2, SUP * 2)], sidx_v,
                                 sem_i),
                pltpu.async_copy(ds_hbm.at[sb_global], dsum_v, sem_i),
            ]
            for cp in cps:
                cp.wait()
            gathers(0, 0)

            def blk(t, carry2):
                ph = t % 2

                @pl.when(t < SUP - 1)
                def _():
                    gathers(t + 1, 1 - ph)

                wait_gathers(t, ph)

                def half(h, carry3):
                    @pl.when(t > 0)
                    def _():
                        wait_scatter(h, 2 * t + h - 2)

                    compute_half(t, h, ph)
                    pltpu.async_copy(
                        out_v.at[pl.ds(h * HALF, HALF)],
                        acc.at[sidx_v.at[2 * t + h]],
                        ssem.at[h], add=True)
                    return carry3

                lax.fori_loop(0, 2, half, 0, unroll=False)
                return carry2

            lax.fori_loop(0, SUP, blk, 0, unroll=False)
            return carry

        lax.fori_loop(0, nsup, superblock, 0, unroll=False)
        wait_scatter(0, 2 * SUP - 2)
        wait_scatter(1, 2 * SUP - 1)

        plsc.subcore_barrier()
        pltpu.sync_copy(acc.at[pl.ds(s * zrows, zrows)],
                        out_hbm.at[c, pl.ds(s * zrows, zrows)])

    return sc_kernel


def kernel(h, pairs_0, pairs_1, degrees_0, degrees_1, scatter_idx,
           W_lin, b_lin, W_t, b_t, eps):
    n, d_in = h.shape
    d_out = W_lin.shape[1]
    e = pairs_0.shape[0]

    iperm = jnp.arange(d_out).reshape(d_out // 32, 2, 16).transpose(
        0, 2, 1).reshape(d_out)
    w_h = W_t[:d_in][:, iperm]
    w_d = W_t[d_in:]
    h3, p_tab = pl.pallas_call(
        _mm_body,
        out_shape=(jax.ShapeDtypeStruct((n, d_out), jnp.float32),
                   jax.ShapeDtypeStruct((n, d_out), jnp.bfloat16)),
    )(h, W_lin, b_lin.reshape(1, d_out),
      w_h, (0.5 * b_t)[iperm].reshape(1, d_out))
    p_i32 = lax.bitcast_convert_type(
        p_tab.reshape(n, d_out // 2, 2), jnp.int32)

    chunk = NW * SUP * BLK
    e_pad = -(-e // chunk) * chunk
    pad = e_pad - e
    zrows = -(-(n + 1) // (NS * 8)) * 8
    n_dump = NS * zrows
    p0 = jnp.pad(pairs_0, (0, pad)).reshape(e_pad // BLK, BLK)
    p1 = jnp.pad(pairs_1, (0, pad)).reshape(e_pad // BLK, BLK)
    si = jnp.pad(scatter_idx, (0, pad), constant_values=n).reshape(
        e_pad // HALF, HALF)
    nsb = e_pad // (SUP * BLK)
    dsum = jnp.pad(degrees_0 + degrees_1, ((0, pad), (0, 0))).T.reshape(
        3, nsb, SUP * BLK).transpose(1, 0, 2)
    zeros = jnp.zeros((zrows, d_out), jnp.float32)

    sc = _make_sc_kernel(n, d_out, e_pad // BLK // NW, n_dump, zrows)
    partials = sc(p_i32, p0, p1, si, dsum, w_d, zeros)

    out = pl.pallas_call(
        functools.partial(_combine_body, n),
        in_specs=[pl.BlockSpec(memory_space=pltpu.VMEM),
                  pl.BlockSpec(memory_space=pltpu.VMEM),
                  pl.BlockSpec(memory_space=pltpu.SMEM)],
        out_shape=jax.ShapeDtypeStruct((n, d_out), jnp.float32),
    )(h3, partials, eps)
    return out


# speedup vs baseline: 1.2237x; 1.0002x over previous
import functools

import jax
import jax.numpy as jnp
from jax import lax
from jax.experimental import pallas as pl
from jax.experimental.pallas import tpu as pltpu
from jax.experimental.pallas import tpu_sc as plsc

LANES = 16
BLK = 64
HALF = 32
SUP = 40
NC, NS = 2, 16
NW = NC * NS


def _mm_body(h_ref, wl_ref, bl_ref, wh_ref, bth_ref, h3_ref, p_ref):
    hb = h_ref[...]
    h3_ref[...] = jnp.dot(hb, wl_ref[...],
                          preferred_element_type=jnp.float32) + bl_ref[...]
    p_ref[...] = (jnp.dot(hb, wh_ref[...],
                          preferred_element_type=jnp.float32)
                  + bth_ref[...]).astype(jnp.bfloat16)


def _combine_body(n, h3_ref, parts_ref, eps_ref, out_ref):
    scale = 1.0 + eps_ref[0]
    sl = pl.ds(0, n)
    out_ref[...] = h3_ref[...] + scale * (parts_ref[0, sl] + parts_ref[1, sl])


def _make_sc_kernel(n_nodes, d, n_blocks_per_worker, n_acc_rows, zrows):
    mesh = plsc.VectorSubcoreMesh(core_axis_name="c", subcore_axis_name="s")
    nsup = n_blocks_per_worker // SUP
    npair = SUP // 2
    nch = d // LANES

    @functools.partial(
        pl.kernel,
        out_type=jax.ShapeDtypeStruct((NC, n_acc_rows, d), jnp.float32),
        mesh=mesh,
        scratch_types=[
            pltpu.VMEM((SUP, BLK), jnp.int32),
            pltpu.VMEM((SUP, BLK), jnp.int32),
            pltpu.VMEM((SUP * 2, HALF), jnp.int32),
            pltpu.VMEM((3, SUP * BLK), jnp.float32),
            pltpu.VMEM((2 * BLK, d // 2), jnp.int32),
            pltpu.VMEM((2 * BLK, d // 2), jnp.int32),
            pltpu.VMEM((2 * HALF, d), jnp.float32),
            pltpu.VMEM((3, d), jnp.float32),
            pltpu.VMEM_SHARED((n_acc_rows, d), jnp.float32),
            pltpu.SemaphoreType.DMA,
            pltpu.SemaphoreType.DMA((2,)),
            pltpu.SemaphoreType.DMA((2,)),
        ],
        compiler_params=pltpu.CompilerParams(use_tc_tiling_on_sc=False),
    )
    def sc_kernel(p_hbm, p0_hbm, p1_hbm, si_hbm, ds_hbm, wd_hbm, z_hbm,
                  out_hbm, idx0_v, idx1_v, sidx_v, dsum_v,
                  r0buf, r1buf, out_v, wd_v, acc, sem_i, gsem, ssem):
        c = lax.axis_index("c")
        s = lax.axis_index("s")
        wid = c * NS + s

        pltpu.sync_copy(z_hbm, acc.at[pl.ds(s * zrows, zrows)])
        pltpu.sync_copy(wd_hbm, wd_v)
        plsc.subcore_barrier()

        def gathers(j, ph):
            sl = pl.ds(ph * BLK, BLK)
            pltpu.async_copy(p_hbm.at[idx0_v.at[j]], r0buf.at[sl],
                             gsem.at[ph])
            pltpu.async_copy(p_hbm.at[idx1_v.at[j]], r1buf.at[sl],
                             gsem.at[ph])

        def wait_gathers(j, ph):
            sl = pl.ds(ph * BLK, BLK)
            pltpu.make_async_copy(p_hbm.at[idx0_v.at[j]], r0buf.at[sl],
                                  gsem.at[ph]).wait()
            pltpu.make_async_copy(p_hbm.at[idx1_v.at[j]], r1buf.at[sl],
                                  gsem.at[ph]).wait()

        def wait_scatter(h, hj):
            pltpu.make_async_copy(out_v.at[pl.ds(h * HALF, HALF)],
                                  acc.at[sidx_v.at[hj]],
                                  ssem.at[h]).wait()

        def compute_half(t, h, ph):
            @plsc.parallel_loop(0, HALF // LANES)
            def group(g):
                wch = [[wd_v[k, pl.ds(ch * LANES, LANES)]
                        for ch in range(nch)] for k in range(3)]
                goff = t * BLK + h * HALF + g * LANES
                dsv = [dsum_v[k, pl.ds(goff, LANES)] for k in range(3)]
                for el in range(LANES):
                    e = ph * BLK + h * HALF + g * LANES + el
                    o = h * HALF + g * LANES + el
                    ds0, ds1, ds2 = dsv[0][el], dsv[1][el], dsv[2][el]
                    for c2 in range(nch // 2):
                        x0 = r0buf[e, pl.ds(c2 * LANES, LANES)]
                        x1 = r1buf[e, pl.ds(c2 * LANES, LANES)]
                        bc = lax.bitcast_convert_type
                        a0 = bc(x0 << 16, jnp.float32)
                        b0 = bc(x0 & jnp.int32(-65536), jnp.float32)
                        a1 = bc(x1 << 16, jnp.float32)
                        b1 = bc(x1 & jnp.int32(-65536), jnp.float32)
                        va = a0 + a1
                        va = va + ds0 * wch[0][2 * c2]
                        va = va + ds1 * wch[1][2 * c2]
                        va = va + ds2 * wch[2][2 * c2]
                        vb = b0 + b1
                        vb = vb + ds0 * wch[0][2 * c2 + 1]
                        vb = vb + ds1 * wch[1][2 * c2 + 1]
                        vb = vb + ds2 * wch[2][2 * c2 + 1]
                        out_v[o, pl.ds(c2 * 2 * LANES, LANES)] = (
                            jnp.maximum(va, 0.0))
                        out_v[o, pl.ds(c2 * 2 * LANES + LANES, LANES)] = (
                            jnp.maximum(vb, 0.0))

        def superblock(sb, carry):
            @pl.when(sb > 0)
            def _():
                wait_scatter(0, 2 * SUP - 2)
                wait_scatter(1, 2 * SUP - 1)

            row0 = (wid * n_blocks_per_worker) + sb * SUP
            sb_global = wid * nsup + sb
            cps = [
                pltpu.async_copy(p0_hbm.at[pl.ds(row0, SUP)], idx0_v, sem_i),
                pltpu.async_copy(p1_hbm.at[pl.ds(row0, SUP)], idx1_v, sem_i),
                pltpu.async_copy(si_hbm.at[pl.ds(row0 * 2, SUP * 2)], sidx_v,
                                 sem_i),
                pltpu.async_copy(ds_hbm.at[sb_global], dsum_v, sem_i),
            ]
            for cp in cps:
                cp.wait()
            gathers(0, 0)

            def blk(t, carry2):
                ph = t % 2

                @pl.when(t < SUP - 1)
                def _():
                    gathers(t + 1, 1 - ph)

                wait_gathers(t, ph)

                def half(h, carry3):
                    @pl.when(t > 0)
                    def _():
                        wait_scatter(h, 2 * t + h - 2)

                    compute_half(t, h, ph)
                    pltpu.async_copy(
                        out_v.at[pl.ds(h * HALF, HALF)],
                        acc.at[sidx_v.at[2 * t + h]],
                        ssem.at[h], add=True)
                    return carry3

                lax.fori_loop(0, 2, half, 0, unroll=False)
                return carry2

            lax.fori_loop(0, SUP, blk, 0, unroll=False)
            return carry

        lax.fori_loop(0, nsup, superblock, 0, unroll=False)
        wait_scatter(0, 2 * SUP - 2)
        wait_scatter(1, 2 * SUP - 1)

        plsc.subcore_barrier()
        pltpu.sync_copy(acc.at[pl.ds(s * zrows, zrows)],
                        out_hbm.at[c, pl.ds(s * zrows, zrows)])

    return sc_kernel


def kernel(h, pairs_0, pairs_1, degrees_0, degrees_1, scatter_idx,
           W_lin, b_lin, W_t, b_t, eps):
    n, d_in = h.shape
    d_out = W_lin.shape[1]
    e = pairs_0.shape[0]

    iperm = jnp.arange(d_out).reshape(d_out // 32, 2, 16).transpose(
        0, 2, 1).reshape(d_out)
    w_h = W_t[:d_in][:, iperm]
    w_d = W_t[d_in:]
    h3, p_tab = pl.pallas_call(
        _mm_body,
        out_shape=(jax.ShapeDtypeStruct((n, d_out), jnp.float32),
                   jax.ShapeDtypeStruct((n, d_out), jnp.bfloat16)),
    )(h, W_lin, b_lin.reshape(1, d_out),
      w_h, (0.5 * b_t)[iperm].reshape(1, d_out))
    p_i32 = lax.bitcast_convert_type(
        p_tab.reshape(n, d_out // 2, 2), jnp.int32)

    chunk = NW * SUP * BLK
    e_pad = -(-e // chunk) * chunk
    pad = e_pad - e
    zrows = -(-(n + 1) // (NS * 8)) * 8
    n_dump = NS * zrows
    p0 = jnp.pad(pairs_0, (0, pad)).reshape(e_pad // BLK, BLK)
    p1 = jnp.pad(pairs_1, (0, pad)).reshape(e_pad // BLK, BLK)
    si = jnp.pad(scatter_idx, (0, pad), constant_values=n).reshape(
        e_pad // HALF, HALF)
    nsb = e_pad // (SUP * BLK)
    dsum = jnp.pad(degrees_0 + degrees_1, ((0, pad), (0, 0))).T.reshape(
        3, nsb, SUP * BLK).transpose(1, 0, 2)
    zeros = jnp.zeros((zrows, d_out), jnp.float32)

    sc = _make_sc_kernel(n, d_out, e_pad // BLK // NW, n_dump, zrows)
    partials = sc(p_i32, p0, p1, si, dsum, w_d, zeros)

    out = pl.pallas_call(
        functools.partial(_combine_body, n),
        in_specs=[pl.BlockSpec(memory_space=pltpu.VMEM),
                  pl.BlockSpec(memory_space=pltpu.VMEM),
                  pl.BlockSpec(memory_space=pltpu.SMEM)],
        out_shape=jax.ShapeDtypeStruct((n, d_out), jnp.float32),
    )(h3, partials, eps)
    return out
